# baseline (speedup 1.0000x reference)
import functools

import jax
import jax.numpy as jnp
import numpy as np
from jax import lax
from jax.experimental import pallas as pl
from jax.experimental.pallas import tpu as pltpu
from jax.experimental.pallas import tpu_sc as plsc

N = 10000
E = 320000
D = 128
NC, NS = 2, 16
NW = NC * NS
CHUNK = 64
NCHUNK = 160
SUP = 40
NBUF = 4
EPW = NCHUNK * CHUNK
E_PAD = NW * EPW
N_PAD = 10240
ROWS_PER_TILE = N_PAD // NS

_MESH = plsc.VectorSubcoreMesh(core_axis_name="c", subcore_axis_name="s")

_PAD_CONST = np.broadcast_to(
    N + np.arange(E_PAD - E, dtype=np.int32) % (N_PAD - N), (2, E_PAD - E))


def _zero16():
    return jnp.zeros((16,), jnp.float32)


def _deg_body(rc_hbm, deg_out, idx_v, ones_v, zbuf_v, deg_sp, sem):
    cid = lax.axis_index("c")
    sid = lax.axis_index("s")
    wid = cid * NS + sid

    def fill_z(i, _):
        zbuf_v[pl.ds(i * 16, 16)] = _zero16()
        return 0

    lax.fori_loop(0, ROWS_PER_TILE // 16, fill_z, 0)

    def fill_one(i, _):
        ones_v[pl.ds(i * 16, 16)] = jnp.ones((16,), jnp.float32)
        return 0

    lax.fori_loop(0, CHUNK // 16, fill_one, 0)

    base = sid * ROWS_PER_TILE
    pltpu.sync_copy(zbuf_v, deg_sp.at[pl.ds(base, ROWS_PER_TILE)])
    pltpu.sync_copy(rc_hbm.at[0, wid], idx_v)
    plsc.subcore_barrier()

    def scat(j, _):
        pltpu.async_copy(ones_v, deg_sp.at[idx_v.at[j]], sem, add=True)
        return 0

    lax.fori_loop(0, NCHUNK, scat, 0)

    def drain(j, _):
        pltpu.make_async_copy(ones_v, deg_sp.at[idx_v.at[j]], sem).wait()
        return 0

    lax.fori_loop(0, NCHUNK, drain, 0)
    plsc.subcore_barrier()

    pltpu.sync_copy(deg_sp.at[pl.ds(base, ROWS_PER_TILE)], zbuf_v)
    pltpu.sync_copy(zbuf_v, deg_out.at[cid, pl.ds(base, ROWS_PER_TILE)])


_deg_kernel = functools.partial(
    pl.kernel,
    out_type=jax.ShapeDtypeStruct((NC, N_PAD), jnp.float32),
    mesh=_MESH,
    scratch_types=[
        pltpu.VMEM((NCHUNK, CHUNK), jnp.int32),
        pltpu.VMEM((CHUNK,), jnp.float32),
        pltpu.VMEM((ROWS_PER_TILE,), jnp.float32),
        pltpu.VMEM_SHARED((N_PAD,), jnp.float32),
        pltpu.SemaphoreType.DMA,
    ],
)(_deg_body)


def _edge_body(y_hbm, rc_hbm, acc_out, idx_r, idx_c, buf0, buf1, buf2, buf3,
               acc_sp, gs0, gs1, gs2, gs3, ss0, ss1, ss2, ss3):
    cid = lax.axis_index("c")
    sid = lax.axis_index("s")
    wid = cid * NS + sid
    bufs = (buf0, buf1, buf2, buf3)
    gsem = (gs0, gs1, gs2, gs3)
    ssem = (ss0, ss1, ss2, ss3)

    def fill_z(i, _):
        r = i // (D // 16)
        c = i % (D // 16)
        buf0[r, pl.ds(c * 16, 16)] = _zero16()
        return 0

    lax.fori_loop(0, CHUNK * (D // 16), fill_z, 0)

    base = sid * ROWS_PER_TILE
    for k in range(ROWS_PER_TILE // CHUNK):
        pltpu.sync_copy(buf0, acc_sp.at[pl.ds(base + k * CHUNK, CHUNK)])

    plsc.subcore_barrier()

    def sup(s, _):
        pltpu.sync_copy(rc_hbm.at[0, wid, pl.ds(s * SUP, SUP)], idx_r)
        pltpu.sync_copy(rc_hbm.at[1, wid, pl.ds(s * SUP, SUP)], idx_c)
        pltpu.async_copy(y_hbm.at[idx_r.at[0]], buf0, gs0)
        pltpu.async_copy(y_hbm.at[idx_r.at[1]], buf1, gs1)

        def step(t, _):
            j0 = NBUF * t
            for k in range(NBUF):
                j = j0 + k
                tgt = (k + 2) % NBUF
                pltpu.make_async_copy(
                    y_hbm.at[idx_r.at[j]], bufs[k], gsem[k]).wait()
                pltpu.async_copy(
                    bufs[k], acc_sp.at[idx_c.at[j]], ssem[k], add=True)

                @pl.when(jnp.logical_and(j + 2 < SUP, j >= 2))
                def _():
                    pltpu.make_async_copy(
                        bufs[tgt], acc_sp.at[idx_c.at[j - 2]],
                        ssem[tgt]).wait()

                @pl.when(j + 2 < SUP)
                def _():
                    pltpu.async_copy(
                        y_hbm.at[idx_r.at[j + 2]], bufs[tgt], gsem[tgt])

            return 0

        lax.fori_loop(0, SUP // NBUF, step, 0)
        for k in range(NBUF):
            j = SUP - NBUF + k
            pltpu.make_async_copy(
                bufs[k], acc_sp.at[idx_c.at[j]], ssem[k]).wait()
        return 0

    lax.fori_loop(0, NCHUNK // SUP, sup, 0)
    plsc.subcore_barrier()

    for k in range(ROWS_PER_TILE // CHUNK):
        pltpu.sync_copy(acc_sp.at[pl.ds(base + k * CHUNK, CHUNK)], buf0)
        pltpu.sync_copy(buf0, acc_out.at[cid, pl.ds(base + k * CHUNK, CHUNK)])


_edge_kernel = functools.partial(
    pl.kernel,
    out_type=jax.ShapeDtypeStruct((NC, N_PAD, D), jnp.float32),
    mesh=_MESH,
    scratch_types=[
        pltpu.VMEM((SUP, CHUNK), jnp.int32),
        pltpu.VMEM((SUP, CHUNK), jnp.int32),
        pltpu.VMEM((CHUNK, D), jnp.float32),
        pltpu.VMEM((CHUNK, D), jnp.float32),
        pltpu.VMEM((CHUNK, D), jnp.float32),
        pltpu.VMEM((CHUNK, D), jnp.float32),
        pltpu.VMEM_SHARED((N_PAD, D), jnp.float32),
        pltpu.SemaphoreType.DMA,
        pltpu.SemaphoreType.DMA,
        pltpu.SemaphoreType.DMA,
        pltpu.SemaphoreType.DMA,
        pltpu.SemaphoreType.DMA,
        pltpu.SemaphoreType.DMA,
        pltpu.SemaphoreType.DMA,
        pltpu.SemaphoreType.DMA,
    ],
)(_edge_body)


def _matmul_body(x_ref, w_ref, xt_ref):
    xt_ref[...] = jnp.dot(x_ref[...], w_ref[...],
                          preferred_element_type=jnp.float32)


def _scale_body(xt_ref, deg2_ref, y_ref, dis_ref):
    deg = deg2_ref[0, :] + deg2_ref[1, :]
    dis = jnp.where(deg > 0.0, lax.rsqrt(deg), 0.0)
    dis_ref[...] = dis
    y_ref[0:N, :] = xt_ref[...] * dis[0:N, None]
    y_ref[N:N_PAD, :] = jnp.zeros((N_PAD - N, D), jnp.float32)


def _finalize_body(acc2_ref, dis_ref, b_ref, out_ref):
    acc = acc2_ref[0, 0:N, :] + acc2_ref[1, 0:N, :]
    out_ref[...] = acc * dis_ref[0:N][:, None] + b_ref[...][None, :]


def kernel(x, edge_index, W, b):
    rc4 = jnp.concatenate([edge_index, _PAD_CONST], axis=1
                          ).reshape(2, NW, NCHUNK, CHUNK)

    deg2 = _deg_kernel(rc4)

    xt = pl.pallas_call(
        _matmul_body,
        out_shape=jax.ShapeDtypeStruct((N, D), jnp.float32),
    )(x, W)

    y, dis = pl.pallas_call(
        _scale_body,
        out_shape=(
            jax.ShapeDtypeStruct((N_PAD, D), jnp.float32),
            jax.ShapeDtypeStruct((N_PAD,), jnp.float32),
        ),
    )(xt, deg2)

    acc2 = _edge_kernel(y, rc4)

    out = pl.pallas_call(
        _finalize_body,
        out_shape=jax.ShapeDtypeStruct((N, D), jnp.float32),
    )(acc2, dis, b)
    return out


# speedup vs baseline: 1.1319x; 1.1319x over previous
import functools

import jax
import jax.numpy as jnp
import numpy as np
from jax import lax
from jax.experimental import pallas as pl
from jax.experimental.pallas import tpu as pltpu
from jax.experimental.pallas import tpu_sc as plsc

N = 10000
E = 320000
D = 128
NC, NS = 2, 16
NW = NC * NS
CHUNK = 128
NCHUNK = 80
SUP = 40
EPW = NCHUNK * CHUNK
E_PAD = NW * EPW
N_PAD = 10240
ROWS_PER_TILE = N_PAD // NS

_MESH = plsc.VectorSubcoreMesh(core_axis_name="c", subcore_axis_name="s")

_PAD_CONST = np.broadcast_to(
    N + np.arange(E_PAD - E, dtype=np.int32) % (N_PAD - N), (2, E_PAD - E))


def _zero16():
    return jnp.zeros((16,), jnp.float32)


def _deg_body(rc_hbm, deg_out, idx_v, ones_v, zbuf_v, deg_sp, sem):
    cid = lax.axis_index("c")
    sid = lax.axis_index("s")
    wid = cid * NS + sid

    def fill_z(i, _):
        zbuf_v[pl.ds(i * 16, 16)] = _zero16()
        return 0

    lax.fori_loop(0, ROWS_PER_TILE // 16, fill_z, 0)

    def fill_one(i, _):
        ones_v[pl.ds(i * 16, 16)] = jnp.ones((16,), jnp.float32)
        return 0

    lax.fori_loop(0, CHUNK // 16, fill_one, 0)

    base = sid * ROWS_PER_TILE
    pltpu.sync_copy(zbuf_v, deg_sp.at[pl.ds(base, ROWS_PER_TILE)])
    pltpu.sync_copy(rc_hbm.at[0, wid], idx_v)
    plsc.subcore_barrier()

    def scat(j, _):
        pltpu.async_copy(ones_v, deg_sp.at[idx_v.at[j]], sem, add=True)
        return 0

    lax.fori_loop(0, NCHUNK, scat, 0)

    def drain(j, _):
        pltpu.make_async_copy(ones_v, deg_sp.at[idx_v.at[j]], sem).wait()
        return 0

    lax.fori_loop(0, NCHUNK, drain, 0)
    plsc.subcore_barrier()

    pltpu.sync_copy(deg_sp.at[pl.ds(base, ROWS_PER_TILE)], zbuf_v)
    pltpu.sync_copy(zbuf_v, deg_out.at[cid, pl.ds(base, ROWS_PER_TILE)])


_deg_kernel = functools.partial(
    pl.kernel,
    out_type=jax.ShapeDtypeStruct((NC, N_PAD), jnp.float32),
    mesh=_MESH,
    scratch_types=[
        pltpu.VMEM((NCHUNK, CHUNK), jnp.int32),
        pltpu.VMEM((CHUNK,), jnp.float32),
        pltpu.VMEM((ROWS_PER_TILE,), jnp.float32),
        pltpu.VMEM_SHARED((N_PAD,), jnp.float32),
        pltpu.SemaphoreType.DMA,
    ],
)(_deg_body)


def _edge_body(y_hbm, rc_hbm, acc_out, idx_r, idx_c, buf0, buf1,
               acc_sp, sem0, sem1):
    cid = lax.axis_index("c")
    sid = lax.axis_index("s")
    wid = cid * NS + sid

    def fill_z(i, _):
        r = i // (D // 16)
        c = i % (D // 16)
        buf0[r, pl.ds(c * 16, 16)] = _zero16()
        return 0

    lax.fori_loop(0, CHUNK * (D // 16), fill_z, 0)

    base = sid * ROWS_PER_TILE
    for k in range(ROWS_PER_TILE // CHUNK):
        pltpu.sync_copy(buf0, acc_sp.at[pl.ds(base + k * CHUNK, CHUNK)])

    plsc.subcore_barrier()

    def sup(s, _):
        pltpu.sync_copy(rc_hbm.at[0, wid, pl.ds(s * SUP, SUP)], idx_r)
        pltpu.sync_copy(rc_hbm.at[1, wid, pl.ds(s * SUP, SUP)], idx_c)
        pltpu.async_copy(y_hbm.at[idx_r.at[0]], buf0, sem0)
        pltpu.async_copy(y_hbm.at[idx_r.at[1]], buf1, sem1)

        def step(t, _):
            j0 = 2 * t
            pltpu.make_async_copy(y_hbm.at[idx_r.at[j0]], buf0, sem0).wait()
            pltpu.sync_copy(buf0, acc_sp.at[idx_c.at[j0]], add=True)

            @pl.when(j0 + 2 < SUP)
            def _():
                pltpu.async_copy(y_hbm.at[idx_r.at[j0 + 2]], buf0, sem0)

            pltpu.make_async_copy(y_hbm.at[idx_r.at[j0 + 1]], buf1, sem1).wait()
            pltpu.sync_copy(buf1, acc_sp.at[idx_c.at[j0 + 1]], add=True)

            @pl.when(j0 + 3 < SUP)
            def _():
                pltpu.async_copy(y_hbm.at[idx_r.at[j0 + 3]], buf1, sem1)

            return 0

        lax.fori_loop(0, SUP // 2, step, 0)
        return 0

    lax.fori_loop(0, NCHUNK // SUP, sup, 0)
    plsc.subcore_barrier()

    for k in range(ROWS_PER_TILE // CHUNK):
        pltpu.sync_copy(acc_sp.at[pl.ds(base + k * CHUNK, CHUNK)], buf0)
        pltpu.sync_copy(buf0, acc_out.at[cid, pl.ds(base + k * CHUNK, CHUNK)])


_edge_kernel = functools.partial(
    pl.kernel,
    out_type=jax.ShapeDtypeStruct((NC, N_PAD, D), jnp.float32),
    mesh=_MESH,
    scratch_types=[
        pltpu.VMEM((SUP, CHUNK), jnp.int32),
        pltpu.VMEM((SUP, CHUNK), jnp.int32),
        pltpu.VMEM((CHUNK, D), jnp.float32),
        pltpu.VMEM((CHUNK, D), jnp.float32),
        pltpu.VMEM_SHARED((N_PAD, D), jnp.float32),
        pltpu.SemaphoreType.DMA,
        pltpu.SemaphoreType.DMA,
    ],
)(_edge_body)


def _matmul_body(x_ref, w_ref, xt_ref):
    xt_ref[...] = jnp.dot(x_ref[...], w_ref[...],
                          preferred_element_type=jnp.float32)


def _scale_body(xt_ref, deg2_ref, y_ref, dis_ref):
    deg = deg2_ref[0, :] + deg2_ref[1, :]
    dis = jnp.where(deg > 0.0, lax.rsqrt(deg), 0.0)
    dis_ref[...] = dis
    y_ref[0:N, :] = xt_ref[...] * dis[0:N, None]
    y_ref[N:N_PAD, :] = jnp.zeros((N_PAD - N, D), jnp.float32)


def _finalize_body(acc2_ref, dis_ref, b_ref, out_ref):
    acc = acc2_ref[0, 0:N, :] + acc2_ref[1, 0:N, :]
    out_ref[...] = acc * dis_ref[0:N][:, None] + b_ref[...][None, :]


def kernel(x, edge_index, W, b):
    rc4 = jnp.concatenate([edge_index, _PAD_CONST], axis=1
                          ).reshape(2, NW, NCHUNK, CHUNK)

    deg2 = _deg_kernel(rc4)

    xt = pl.pallas_call(
        _matmul_body,
        out_shape=jax.ShapeDtypeStruct((N, D), jnp.float32),
    )(x, W)

    y, dis = pl.pallas_call(
        _scale_body,
        out_shape=(
            jax.ShapeDtypeStruct((N_PAD, D), jnp.float32),
            jax.ShapeDtypeStruct((N_PAD,), jnp.float32),
        ),
    )(xt, deg2)

    acc2 = _edge_kernel(y, rc4)

    out = pl.pallas_call(
        _finalize_body,
        out_shape=jax.ShapeDtypeStruct((N, D), jnp.float32),
    )(acc2, dis, b)
    return out


# speedup vs baseline: 1.1422x; 1.0091x over previous
import functools

import jax
import jax.numpy as jnp
import numpy as np
from jax import lax
from jax.experimental import pallas as pl
from jax.experimental.pallas import tpu as pltpu
from jax.experimental.pallas import tpu_sc as plsc

N = 10000
E = 320000
D = 128
NC, NS = 2, 16
NW = NC * NS
CHUNK = 128
NCHUNK = 80
SUP = 40
EPW = NCHUNK * CHUNK
E_PAD = NW * EPW
N_PAD = 10240
ROWS_PER_TILE = N_PAD // NS

_MESH = plsc.VectorSubcoreMesh(core_axis_name="c", subcore_axis_name="s")

_PAD_CONST = np.broadcast_to(
    N + np.arange(E_PAD - E, dtype=np.int32) % (N_PAD - N), (2, E_PAD - E))


def _zero16():
    return jnp.zeros((16,), jnp.float32)


def _deg_body(rc_hbm, deg_out, idx_v, ones_v, zbuf_v, deg_sp, sem):
    cid = lax.axis_index("c")
    sid = lax.axis_index("s")
    wid = cid * NS + sid

    def fill_z(i, _):
        zbuf_v[pl.ds(i * 16, 16)] = _zero16()
        return 0

    lax.fori_loop(0, ROWS_PER_TILE // 16, fill_z, 0)

    def fill_one(i, _):
        ones_v[pl.ds(i * 16, 16)] = jnp.ones((16,), jnp.float32)
        return 0

    lax.fori_loop(0, CHUNK // 16, fill_one, 0)

    base = sid * ROWS_PER_TILE
    pltpu.sync_copy(zbuf_v, deg_sp.at[pl.ds(base, ROWS_PER_TILE)])
    pltpu.sync_copy(rc_hbm.at[0, wid], idx_v)
    plsc.subcore_barrier()

    def scat(j, _):
        pltpu.async_copy(ones_v, deg_sp.at[idx_v.at[j]], sem, add=True)
        return 0

    lax.fori_loop(0, NCHUNK, scat, 0)

    def drain(j, _):
        pltpu.make_async_copy(ones_v, deg_sp.at[idx_v.at[j]], sem).wait()
        return 0

    lax.fori_loop(0, NCHUNK, drain, 0)
    plsc.subcore_barrier()

    pltpu.sync_copy(deg_sp.at[pl.ds(base, ROWS_PER_TILE)], zbuf_v)
    pltpu.sync_copy(zbuf_v, deg_out.at[cid, pl.ds(base, ROWS_PER_TILE)])


_deg_kernel = functools.partial(
    pl.kernel,
    out_type=jax.ShapeDtypeStruct((NC, N_PAD), jnp.float32),
    mesh=_MESH,
    scratch_types=[
        pltpu.VMEM((NCHUNK, CHUNK), jnp.int32),
        pltpu.VMEM((CHUNK,), jnp.float32),
        pltpu.VMEM((ROWS_PER_TILE,), jnp.float32),
        pltpu.VMEM_SHARED((N_PAD,), jnp.float32),
        pltpu.SemaphoreType.DMA,
    ],
)(_deg_body)


def _edge_body(y_hbm, rc_hbm, acc_out, idx_r, idx_c, buf0, buf1,
               acc_sp, sem0, sem1):
    cid = lax.axis_index("c")
    sid = lax.axis_index("s")
    wid = cid * NS + sid

    def fill_z(i, _):
        r = i // (D // 16)
        c = i % (D // 16)
        buf0[r, pl.ds(c * 16, 16)] = _zero16()
        return 0

    lax.fori_loop(0, CHUNK * (D // 16), fill_z, 0)

    base = sid * ROWS_PER_TILE
    for k in range(ROWS_PER_TILE // CHUNK):
        pltpu.sync_copy(buf0, acc_sp.at[pl.ds(base + k * CHUNK, CHUNK)])

    plsc.subcore_barrier()

    def sup(s, _):
        pltpu.sync_copy(rc_hbm.at[0, wid, pl.ds(s * SUP, SUP)], idx_r)
        pltpu.sync_copy(rc_hbm.at[1, wid, pl.ds(s * SUP, SUP)], idx_c)
        pltpu.async_copy(y_hbm.at[idx_r.at[0]], buf0, sem0)
        pltpu.async_copy(y_hbm.at[idx_r.at[1]], buf1, sem1)

        def step(t, _):
            j0 = 2 * t
            pltpu.make_async_copy(y_hbm.at[idx_r.at[j0]], buf0, sem0).wait()
            pltpu.sync_copy(buf0, acc_sp.at[idx_c.at[j0]], add=True)

            @pl.when(j0 + 2 < SUP)
            def _():
                pltpu.async_copy(y_hbm.at[idx_r.at[j0 + 2]], buf0, sem0)

            pltpu.make_async_copy(y_hbm.at[idx_r.at[j0 + 1]], buf1, sem1).wait()
            pltpu.sync_copy(buf1, acc_sp.at[idx_c.at[j0 + 1]], add=True)

            @pl.when(j0 + 3 < SUP)
            def _():
                pltpu.async_copy(y_hbm.at[idx_r.at[j0 + 3]], buf1, sem1)

            return 0

        lax.fori_loop(0, SUP // 2, step, 0)
        return 0

    lax.fori_loop(0, NCHUNK // SUP, sup, 0)
    plsc.subcore_barrier()

    flush_bufs = (buf0, buf1)
    flush_sems = (sem0, sem1)
    for k in range(ROWS_PER_TILE // CHUNK):
        fb, fs = flush_bufs[k % 2], flush_sems[k % 2]
        sl = pl.ds(base + k * CHUNK, CHUNK)
        if k >= 2:
            prev = pl.ds(base + (k - 2) * CHUNK, CHUNK)
            pltpu.make_async_copy(fb, acc_out.at[cid, prev], fs).wait()
        pltpu.sync_copy(acc_sp.at[sl], fb)
        pltpu.async_copy(fb, acc_out.at[cid, sl], fs)
    for k in range(ROWS_PER_TILE // CHUNK - 2, ROWS_PER_TILE // CHUNK):
        fb, fs = flush_bufs[k % 2], flush_sems[k % 2]
        sl = pl.ds(base + k * CHUNK, CHUNK)
        pltpu.make_async_copy(fb, acc_out.at[cid, sl], fs).wait()


_edge_kernel = functools.partial(
    pl.kernel,
    out_type=jax.ShapeDtypeStruct((NC, N_PAD, D), jnp.float32),
    mesh=_MESH,
    scratch_types=[
        pltpu.VMEM((SUP, CHUNK), jnp.int32),
        pltpu.VMEM((SUP, CHUNK), jnp.int32),
        pltpu.VMEM((CHUNK, D), jnp.float32),
        pltpu.VMEM((CHUNK, D), jnp.float32),
        pltpu.VMEM_SHARED((N_PAD, D), jnp.float32),
        pltpu.SemaphoreType.DMA,
        pltpu.SemaphoreType.DMA,
    ],
)(_edge_body)


def _matmul_body(x_ref, w_ref, xt_ref):
    xt_ref[...] = jnp.dot(x_ref[...], w_ref[...],
                          preferred_element_type=jnp.float32)


def _scale_body(xt_ref, deg2_ref, y_ref, dis_ref):
    deg = deg2_ref[0, :] + deg2_ref[1, :]
    dis = jnp.where(deg > 0.0, lax.rsqrt(deg), 0.0)
    dis_ref[...] = dis
    y_ref[0:N, :] = xt_ref[...] * dis[0:N, None]
    y_ref[N:N_PAD, :] = jnp.zeros((N_PAD - N, D), jnp.float32)


def _finalize_body(acc2_ref, dis_ref, b_ref, out_ref):
    acc = acc2_ref[0, 0:N, :] + acc2_ref[1, 0:N, :]
    out_ref[...] = acc * dis_ref[0:N][:, None] + b_ref[...][None, :]


def kernel(x, edge_index, W, b):
    rc4 = jnp.concatenate([edge_index, _PAD_CONST], axis=1
                          ).reshape(2, NW, NCHUNK, CHUNK)

    deg2 = _deg_kernel(rc4)

    xt = pl.pallas_call(
        _matmul_body,
        out_shape=jax.ShapeDtypeStruct((N, D), jnp.float32),
    )(x, W)

    y, dis = pl.pallas_call(
        _scale_body,
        out_shape=(
            jax.ShapeDtypeStruct((N_PAD, D), jnp.float32),
            jax.ShapeDtypeStruct((N_PAD,), jnp.float32),
        ),
    )(xt, deg2)

    acc2 = _edge_kernel(y, rc4)

    out = pl.pallas_call(
        _finalize_body,
        out_shape=jax.ShapeDtypeStruct((N, D), jnp.float32),
    )(acc2, dis, b)
    return out
